# baseline (speedup 1.0000x reference)
import functools

import jax
import jax.numpy as jnp
from jax import lax
from jax.experimental import pallas as pl
from jax.experimental.pallas import tpu as pltpu
from jax.experimental.pallas import tpu_sc as plsc

_N_EDGES = 320000
_N_NODES = 10000
_D = 128

_NC = 2
_NS = 16
_NW = _NC * _NS

_EBLK = 2560


def _mm1_body(x_ref, w_ref, b_ref, z_ref):
    acc = jnp.dot(x_ref[...], w_ref[...], preferred_element_type=jnp.float32)
    z_ref[...] = acc + b_ref[...]


def _mm1(x, w1, b):
    return pl.pallas_call(
        _mm1_body,
        grid=(_N_EDGES // _EBLK,),
        in_specs=[
            pl.BlockSpec((_EBLK, _D), lambda i: (i, 0)),
            pl.BlockSpec((_D, _D), lambda i: (0, 0)),
            pl.BlockSpec((1, _D), lambda i: (0, 0)),
        ],
        out_specs=pl.BlockSpec((_EBLK, _D), lambda i: (i, 0)),
        out_shape=jax.ShapeDtypeStruct((_N_EDGES, _D), jnp.float32),
    )(x, w1, b)


_EBLK2 = 2000


def _mm2_add_body(x_ref, w_ref, ye_ref, o_ref):
    acc = jnp.dot(x_ref[...], w_ref[...], preferred_element_type=jnp.float32)
    o_ref[...] = acc + ye_ref[...]


def _mm2_add(x, w2a, y_exp):
    n = x.shape[0]
    return pl.pallas_call(
        _mm2_add_body,
        grid=(n // _EBLK2,),
        in_specs=[
            pl.BlockSpec((_EBLK2, _D), lambda i: (i, 0)),
            pl.BlockSpec((_D, _D), lambda i: (0, 0)),
            pl.BlockSpec((_EBLK2, _D), lambda i: (i, 0)),
        ],
        out_specs=pl.BlockSpec((_EBLK2, _D), lambda i: (i, 0)),
        out_shape=jax.ShapeDtypeStruct((n, _D), jnp.float32),
    )(x, w2a, y_exp)


_NBLK = 2000


def _mm_small_body(zm_ref, w_ref, y_ref):
    y_ref[...] = jnp.dot(zm_ref[...], w_ref[...], preferred_element_type=jnp.float32)


def _mm_small(zmax, w2b):
    return pl.pallas_call(
        _mm_small_body,
        grid=(_N_NODES // _NBLK,),
        in_specs=[
            pl.BlockSpec((_NBLK, _D), lambda i: (i, 0)),
            pl.BlockSpec((_D, _D), lambda i: (0, 0)),
        ],
        out_specs=pl.BlockSpec((_NBLK, _D), lambda i: (i, 0)),
        out_shape=jax.ShapeDtypeStruct((_N_NODES, _D), jnp.float32),
    )(zmax, w2b)


_CHUNK = _N_EDGES // _NW
_SB = 400
_NREG = _D // 16

_sc_mesh = plsc.VectorSubcoreMesh(core_axis_name="c", subcore_axis_name="s")


_NEG = float("-inf")


@functools.partial(
    pl.kernel,
    out_type=jax.ShapeDtypeStruct((_N_NODES * _D,), jnp.float32),
    mesh=_sc_mesh,
    scratch_types=[
        pltpu.VMEM((_SB,), jnp.int32),
        pltpu.VMEM((_SB * _D,), jnp.float32),
        pltpu.VMEM((16,), jnp.int32),
        pltpu.VMEM((_D,), jnp.float32),
        pltpu.VMEM((_D,), jnp.float32),
        pltpu.VMEM((16,), jnp.int32),
        pltpu.VMEM((16,), jnp.int32),
        pltpu.VMEM((16,), jnp.int32),
        pltpu.VMEM((16,), jnp.int32),
        pltpu.VMEM((_D,), jnp.float32),
        pltpu.SemaphoreType.DMA,
        pltpu.SemaphoreType.DMA,
    ],
)
def _segmax_kernel(z_hbm, vid_hbm, zmax_hbm, ids_v, z_v, prev_v, row0, row1,
                   own_v, cur_v, prevg_v, fcnt_v, acc_v, sf0, sf1):
    wid = lax.axis_index("s") * _NC + lax.axis_index("c")
    start = wid * _CHUNK
    chunk_end = start + _CHUNK
    nblk = (_N_EDGES - start) // _SB

    @pl.when(wid > 0)
    def _():
        pltpu.sync_copy(vid_hbm.at[pl.ds(start - 16, 16)], prev_v)

    prev0 = jnp.where(wid > 0, prev_v[...][15], -1)

    def flush_slot(row_v, sem, fcnt, cur, a):
        @pl.when(fcnt >= 2)
        def _():
            pltpu.make_async_copy(row_v, zmax_hbm.at[pl.ds(0, _D)], sem).wait()
        for k in range(_NREG):
            row_v[pl.ds(16 * k, 16)] = a[k]
        pltpu.async_copy(row_v, zmax_hbm.at[pl.ds(cur * _D, _D)], sem)

    def one_edge(ge, base, eid, c):
        own, cur, prev, fcnt = c[0], c[1], c[2], c[3]
        a = c[4:]
        row = tuple(z_v[pl.ds(base + 16 * k, 16)] for k in range(_NREG))
        in_chunk = ge < chunk_end
        b = eid != prev
        flush_now = b & (own == 1)

        @pl.when(flush_now)
        def _():
            @pl.when(lax.rem(fcnt, 2) == 0)
            def _():
                flush_slot(row0, sf0, fcnt, cur, a)

            @pl.when(lax.rem(fcnt, 2) == 1)
            def _():
                flush_slot(row1, sf1, fcnt, cur, a)

        new_own = jnp.where(
            b, jnp.where(in_chunk, 1, 0), own
        )
        new_cur = jnp.where(b, eid, cur)
        new_fcnt = fcnt + jnp.where(flush_now, 1, 0)
        new_a = tuple(
            jnp.maximum(jnp.where(b, _NEG, a[k]), row[k])
            for k in range(_NREG)
        )
        return (new_own, new_cur, eid, new_fcnt) + new_a

    own_v[...] = jnp.full((16,), 0, jnp.int32)
    cur_v[...] = jnp.full((16,), -1, jnp.int32)
    prevg_v[...] = jnp.full((16,), prev0, jnp.int32)
    fcnt_v[...] = jnp.full((16,), 0, jnp.int32)

    def wbody(g, carry):
        own0 = own_v[...][0]
        blk = start + g * _SB

        @pl.when((blk < chunk_end) | (own0 == 1))
        def _():
            pltpu.sync_copy(vid_hbm.at[pl.ds(blk, _SB)], ids_v)
            pltpu.sync_copy(z_hbm.at[pl.ds(blk * _D, _SB * _D)], z_v)
            cur0 = cur_v[...][0]
            prevg0 = prevg_v[...][0]
            fcnt0 = fcnt_v[...][0]
            a0 = tuple(acc_v[pl.ds(16 * k, 16)] for k in range(_NREG))

            def group_step(gi, c):
                idvec = ids_v[pl.ds(gi * 16, 16)]
                for l in range(16):
                    c = one_edge(
                        blk + gi * 16 + l, (gi * 16 + l) * _D, idvec[l], c
                    )
                return c

            res = lax.fori_loop(
                0, _SB // 16, group_step, (own0, cur0, prevg0, fcnt0) + a0
            )
            own_v[...] = jnp.full((16,), res[0], jnp.int32)
            cur_v[...] = jnp.full((16,), res[1], jnp.int32)
            prevg_v[...] = jnp.full((16,), res[2], jnp.int32)
            fcnt_v[...] = jnp.full((16,), res[3], jnp.int32)
            for k in range(_NREG):
                acc_v[pl.ds(16 * k, 16)] = res[4 + k]

        return carry

    lax.fori_loop(0, nblk, wbody, 0)

    own = own_v[...][0]
    cur = cur_v[...][0]

    @pl.when(own == 1)
    def _():
        pltpu.sync_copy(acc_v, zmax_hbm.at[pl.ds(cur * _D, _D)])

    fcnt = fcnt_v[...][0]

    @pl.when(fcnt >= 1)
    def _():
        last = lax.rem(fcnt - 1, 2)

        @pl.when(last == 0)
        def _():
            pltpu.make_async_copy(row0, zmax_hbm.at[pl.ds(0, _D)], sf0).wait()

        @pl.when(last == 1)
        def _():
            pltpu.make_async_copy(row1, zmax_hbm.at[pl.ds(0, _D)], sf1).wait()

    @pl.when(fcnt >= 2)
    def _():
        last2 = lax.rem(fcnt - 2, 2)

        @pl.when(last2 == 0)
        def _():
            pltpu.make_async_copy(row0, zmax_hbm.at[pl.ds(0, _D)], sf0).wait()

        @pl.when(last2 == 1)
        def _():
            pltpu.make_async_copy(row1, zmax_hbm.at[pl.ds(0, _D)], sf1).wait()


_GB = 200


def _make_pump(n_slice):
    chunk = n_slice // _NW
    tpw = chunk // _GB
    pairs = tpw // 2

    @functools.partial(
        pl.kernel,
        out_type=jax.ShapeDtypeStruct((n_slice, _D), jnp.float32),
        mesh=_sc_mesh,
        scratch_types=[
            pltpu.VMEM((_GB,), jnp.int32),
            pltpu.VMEM((_GB,), jnp.int32),
            pltpu.VMEM((_GB, _D), jnp.float32),
            pltpu.VMEM((_GB, _D), jnp.float32),
            pltpu.SemaphoreType.DMA,
            pltpu.SemaphoreType.DMA,
            pltpu.SemaphoreType.DMA,
            pltpu.SemaphoreType.DMA,
        ],
    )
    def _pump(vid_hbm, y_hbm, out_hbm, idx0, idx1, rows0, rows1,
              sg0, sg1, sw0, sw1):
        wid = lax.axis_index("s") * _NC + lax.axis_index("c")
        start = wid * chunk

        def pair(i, carry):
            b0 = start + (2 * i) * _GB
            b1 = start + (2 * i + 1) * _GB

            @pl.when(i > 0)
            def _():
                pltpu.make_async_copy(
                    rows0, out_hbm.at[pl.ds(b0, _GB)], sw0).wait()
                pltpu.make_async_copy(
                    rows1, out_hbm.at[pl.ds(b1, _GB)], sw1).wait()

            pltpu.sync_copy(vid_hbm.at[pl.ds(b0, _GB)], idx0)
            pltpu.async_copy(y_hbm.at[idx0], rows0, sg0)
            pltpu.sync_copy(vid_hbm.at[pl.ds(b1, _GB)], idx1)
            pltpu.async_copy(y_hbm.at[idx1], rows1, sg1)

            pltpu.make_async_copy(y_hbm.at[idx0], rows0, sg0).wait()
            pltpu.async_copy(rows0, out_hbm.at[pl.ds(b0, _GB)], sw0)
            pltpu.make_async_copy(y_hbm.at[idx1], rows1, sg1).wait()
            pltpu.async_copy(rows1, out_hbm.at[pl.ds(b1, _GB)], sw1)
            return carry

        lax.fori_loop(0, pairs, pair, 0)
        if tpw % 2 == 1:
            bt = start + (tpw - 1) * _GB
            if pairs > 0:
                pltpu.make_async_copy(
                    rows0, out_hbm.at[pl.ds(bt - 2 * _GB, _GB)], sw0).wait()
            pltpu.sync_copy(vid_hbm.at[pl.ds(bt, _GB)], idx0)
            pltpu.async_copy(y_hbm.at[idx0], rows0, sg0)
            pltpu.make_async_copy(y_hbm.at[idx0], rows0, sg0).wait()
            pltpu.async_copy(rows0, out_hbm.at[pl.ds(bt, _GB)], sw0)
            if pairs > 0:
                pltpu.make_async_copy(
                    rows1, out_hbm.at[pl.ds(bt - _GB, _GB)], sw1).wait()
            pltpu.make_async_copy(rows0, out_hbm.at[pl.ds(bt, _GB)], sw0).wait()
        else:
            end0 = start + (tpw - 2) * _GB
            end1 = start + (tpw - 1) * _GB
            pltpu.make_async_copy(rows0, out_hbm.at[pl.ds(end0, _GB)], sw0).wait()
            pltpu.make_async_copy(rows1, out_hbm.at[pl.ds(end1, _GB)], sw1).wait()

    return _pump


_N_SLICES = 2
_SLICE = _N_EDGES // _N_SLICES
_pump_slice = _make_pump(_SLICE)


def kernel(x, vertex_id, W1, b1, W2):
    w2a = W2[:_D]
    w2b = W2[_D:]
    z = _mm1(x, W1, b1.reshape(1, _D))
    zmax_flat = _segmax_kernel(z.reshape(-1), vertex_id)
    y = _mm_small(zmax_flat.reshape(_N_NODES, _D), w2b)
    outs = []
    for s in range(_N_SLICES):
        lo = s * _SLICE
        y_exp = _pump_slice(vertex_id[lo:lo + _SLICE], y)
        outs.append(_mm2_add(x[lo:lo + _SLICE], w2a, y_exp))
    return jnp.concatenate(outs, axis=0)


# speedup vs baseline: 1.1615x; 1.1615x over previous
import functools

import jax
import jax.numpy as jnp
from jax import lax
from jax.experimental import pallas as pl
from jax.experimental.pallas import tpu as pltpu
from jax.experimental.pallas import tpu_sc as plsc

_N_EDGES = 320000
_N_NODES = 10000
_D = 128

_NC = 2
_NS = 16
_NW = _NC * _NS

_EBLK = 2560


def _mm1_body(x_ref, w_ref, b_ref, z_ref):
    acc = jnp.dot(x_ref[...], w_ref[...], preferred_element_type=jnp.float32)
    z_ref[...] = acc + b_ref[...]


def _mm1(x, w1, b):
    return pl.pallas_call(
        _mm1_body,
        grid=(_N_EDGES // _EBLK,),
        in_specs=[
            pl.BlockSpec((_EBLK, _D), lambda i: (i, 0)),
            pl.BlockSpec((_D, _D), lambda i: (0, 0)),
            pl.BlockSpec((1, _D), lambda i: (0, 0)),
        ],
        out_specs=pl.BlockSpec((_EBLK, _D), lambda i: (i, 0)),
        out_shape=jax.ShapeDtypeStruct((_N_EDGES, _D), jnp.float32),
    )(x, w1, b)


_EBLK2 = 2000


def _mm2_add_body(x_ref, w_ref, ye_ref, o_ref):
    acc = jnp.dot(x_ref[...], w_ref[...], preferred_element_type=jnp.float32)
    o_ref[...] = acc + ye_ref[...]


def _mm2_add(x, w2a, y_exp):
    n = x.shape[0]
    return pl.pallas_call(
        _mm2_add_body,
        grid=(n // _EBLK2,),
        in_specs=[
            pl.BlockSpec((_EBLK2, _D), lambda i: (i, 0)),
            pl.BlockSpec((_D, _D), lambda i: (0, 0)),
            pl.BlockSpec((_EBLK2, _D), lambda i: (i, 0)),
        ],
        out_specs=pl.BlockSpec((_EBLK2, _D), lambda i: (i, 0)),
        out_shape=jax.ShapeDtypeStruct((n, _D), jnp.float32),
    )(x, w2a, y_exp)


_NBLK = 2000


def _mm_small_body(zm_ref, w_ref, y_ref):
    y_ref[...] = jnp.dot(zm_ref[...], w_ref[...], preferred_element_type=jnp.float32)


def _mm_small(zmax, w2b):
    return pl.pallas_call(
        _mm_small_body,
        grid=(_N_NODES // _NBLK,),
        in_specs=[
            pl.BlockSpec((_NBLK, _D), lambda i: (i, 0)),
            pl.BlockSpec((_D, _D), lambda i: (0, 0)),
        ],
        out_specs=pl.BlockSpec((_NBLK, _D), lambda i: (i, 0)),
        out_shape=jax.ShapeDtypeStruct((_N_NODES, _D), jnp.float32),
    )(zmax, w2b)


_CHUNK = _N_EDGES // _NW
_SB = 400
_NREG = _D // 16

_sc_mesh = plsc.VectorSubcoreMesh(core_axis_name="c", subcore_axis_name="s")


_NEG = float("-inf")


@functools.partial(
    pl.kernel,
    out_type=jax.ShapeDtypeStruct((_N_NODES * _D,), jnp.float32),
    mesh=_sc_mesh,
    scratch_types=[
        pltpu.VMEM((_SB,), jnp.int32),
        pltpu.VMEM((_SB * _D,), jnp.float32),
        pltpu.VMEM((16,), jnp.int32),
        pltpu.VMEM((_D,), jnp.float32),
        pltpu.VMEM((_D,), jnp.float32),
        pltpu.VMEM((16,), jnp.int32),
        pltpu.VMEM((16,), jnp.int32),
        pltpu.VMEM((16,), jnp.int32),
        pltpu.VMEM((16,), jnp.int32),
        pltpu.VMEM((_D,), jnp.float32),
        pltpu.SemaphoreType.DMA,
        pltpu.SemaphoreType.DMA,
    ],
)
def _segmax_kernel(z_hbm, vid_hbm, zmax_hbm, ids_v, z_v, prev_v, row0, row1,
                   own_v, cur_v, prevg_v, fcnt_v, acc_v, sf0, sf1):
    wid = lax.axis_index("s") * _NC + lax.axis_index("c")
    start = wid * _CHUNK
    chunk_end = start + _CHUNK
    nblk = (_N_EDGES - start) // _SB

    @pl.when(wid > 0)
    def _():
        pltpu.sync_copy(vid_hbm.at[pl.ds(start - 16, 16)], prev_v)

    prev0 = jnp.where(wid > 0, prev_v[...][15], -1)

    def flush_slot(row_v, sem, fcnt, cur, a):
        @pl.when(fcnt >= 2)
        def _():
            pltpu.make_async_copy(row_v, zmax_hbm.at[pl.ds(0, _D)], sem).wait()
        for k in range(_NREG):
            row_v[pl.ds(16 * k, 16)] = a[k]
        pltpu.async_copy(row_v, zmax_hbm.at[pl.ds(cur * _D, _D)], sem)

    def one_edge(ge, base, eid, c):
        own, cur, prev, fcnt = c[0], c[1], c[2], c[3]
        a = c[4:]
        row = tuple(z_v[pl.ds(base + 16 * k, 16)] for k in range(_NREG))
        in_chunk = ge < chunk_end
        b = eid != prev
        flush_now = b & (own == 1)

        @pl.when(flush_now)
        def _():
            @pl.when(lax.rem(fcnt, 2) == 0)
            def _():
                flush_slot(row0, sf0, fcnt, cur, a)

            @pl.when(lax.rem(fcnt, 2) == 1)
            def _():
                flush_slot(row1, sf1, fcnt, cur, a)

        new_own = jnp.where(
            b, jnp.where(in_chunk, 1, 0), own
        )
        new_cur = jnp.where(b, eid, cur)
        new_fcnt = fcnt + jnp.where(flush_now, 1, 0)
        new_a = tuple(
            jnp.maximum(jnp.where(b, _NEG, a[k]), row[k])
            for k in range(_NREG)
        )
        return (new_own, new_cur, eid, new_fcnt) + new_a

    own_v[...] = jnp.full((16,), 0, jnp.int32)
    cur_v[...] = jnp.full((16,), -1, jnp.int32)
    prevg_v[...] = jnp.full((16,), prev0, jnp.int32)
    fcnt_v[...] = jnp.full((16,), 0, jnp.int32)

    def wbody(g, carry):
        own0 = own_v[...][0]
        blk = start + g * _SB

        @pl.when((blk < chunk_end) | (own0 == 1))
        def _():
            pltpu.sync_copy(vid_hbm.at[pl.ds(blk, _SB)], ids_v)
            pltpu.sync_copy(z_hbm.at[pl.ds(blk * _D, _SB * _D)], z_v)
            cur0 = cur_v[...][0]
            prevg0 = prevg_v[...][0]
            fcnt0 = fcnt_v[...][0]
            a0 = tuple(acc_v[pl.ds(16 * k, 16)] for k in range(_NREG))

            def group_step(gi, c):
                idvec = ids_v[pl.ds(gi * 16, 16)]
                for l in range(16):
                    c = one_edge(
                        blk + gi * 16 + l, (gi * 16 + l) * _D, idvec[l], c
                    )
                return c

            res = lax.fori_loop(
                0, _SB // 16, group_step, (own0, cur0, prevg0, fcnt0) + a0
            )
            own_v[...] = jnp.full((16,), res[0], jnp.int32)
            cur_v[...] = jnp.full((16,), res[1], jnp.int32)
            prevg_v[...] = jnp.full((16,), res[2], jnp.int32)
            fcnt_v[...] = jnp.full((16,), res[3], jnp.int32)
            for k in range(_NREG):
                acc_v[pl.ds(16 * k, 16)] = res[4 + k]

        return carry

    lax.fori_loop(0, nblk, wbody, 0)

    own = own_v[...][0]
    cur = cur_v[...][0]

    @pl.when(own == 1)
    def _():
        pltpu.sync_copy(acc_v, zmax_hbm.at[pl.ds(cur * _D, _D)])

    fcnt = fcnt_v[...][0]

    @pl.when(fcnt >= 1)
    def _():
        last = lax.rem(fcnt - 1, 2)

        @pl.when(last == 0)
        def _():
            pltpu.make_async_copy(row0, zmax_hbm.at[pl.ds(0, _D)], sf0).wait()

        @pl.when(last == 1)
        def _():
            pltpu.make_async_copy(row1, zmax_hbm.at[pl.ds(0, _D)], sf1).wait()

    @pl.when(fcnt >= 2)
    def _():
        last2 = lax.rem(fcnt - 2, 2)

        @pl.when(last2 == 0)
        def _():
            pltpu.make_async_copy(row0, zmax_hbm.at[pl.ds(0, _D)], sf0).wait()

        @pl.when(last2 == 1)
        def _():
            pltpu.make_async_copy(row1, zmax_hbm.at[pl.ds(0, _D)], sf1).wait()


_GB = 200


def _make_pump(n_slice):
    chunk = n_slice // _NW
    tpw = chunk // _GB
    pairs = tpw // 2

    @functools.partial(
        pl.kernel,
        out_type=jax.ShapeDtypeStruct((n_slice, _D), jnp.float32),
        mesh=_sc_mesh,
        scratch_types=[
            pltpu.VMEM((_GB,), jnp.int32),
            pltpu.VMEM((_GB,), jnp.int32),
            pltpu.VMEM((_GB, _D), jnp.float32),
            pltpu.VMEM((_GB, _D), jnp.float32),
            pltpu.SemaphoreType.DMA,
            pltpu.SemaphoreType.DMA,
            pltpu.SemaphoreType.DMA,
            pltpu.SemaphoreType.DMA,
        ],
    )
    def _pump(vid_hbm, y_hbm, out_hbm, idx0, idx1, rows0, rows1,
              sg0, sg1, sw0, sw1):
        wid = lax.axis_index("s") * _NC + lax.axis_index("c")
        start = wid * chunk

        def pair(i, carry):
            b0 = start + (2 * i) * _GB
            b1 = start + (2 * i + 1) * _GB

            @pl.when(i > 0)
            def _():
                pltpu.make_async_copy(
                    rows0, out_hbm.at[pl.ds(b0, _GB)], sw0).wait()
                pltpu.make_async_copy(
                    rows1, out_hbm.at[pl.ds(b1, _GB)], sw1).wait()

            pltpu.sync_copy(vid_hbm.at[pl.ds(b0, _GB)], idx0)
            pltpu.async_copy(y_hbm.at[idx0], rows0, sg0)
            pltpu.sync_copy(vid_hbm.at[pl.ds(b1, _GB)], idx1)
            pltpu.async_copy(y_hbm.at[idx1], rows1, sg1)

            pltpu.make_async_copy(y_hbm.at[idx0], rows0, sg0).wait()
            pltpu.async_copy(rows0, out_hbm.at[pl.ds(b0, _GB)], sw0)
            pltpu.make_async_copy(y_hbm.at[idx1], rows1, sg1).wait()
            pltpu.async_copy(rows1, out_hbm.at[pl.ds(b1, _GB)], sw1)
            return carry

        lax.fori_loop(0, pairs, pair, 0)
        if tpw % 2 == 1:
            bt = start + (tpw - 1) * _GB
            if pairs > 0:
                pltpu.make_async_copy(
                    rows0, out_hbm.at[pl.ds(bt - 2 * _GB, _GB)], sw0).wait()
            pltpu.sync_copy(vid_hbm.at[pl.ds(bt, _GB)], idx0)
            pltpu.async_copy(y_hbm.at[idx0], rows0, sg0)
            pltpu.make_async_copy(y_hbm.at[idx0], rows0, sg0).wait()
            pltpu.async_copy(rows0, out_hbm.at[pl.ds(bt, _GB)], sw0)
            if pairs > 0:
                pltpu.make_async_copy(
                    rows1, out_hbm.at[pl.ds(bt - _GB, _GB)], sw1).wait()
            pltpu.make_async_copy(rows0, out_hbm.at[pl.ds(bt, _GB)], sw0).wait()
        else:
            end0 = start + (tpw - 2) * _GB
            end1 = start + (tpw - 1) * _GB
            pltpu.make_async_copy(rows0, out_hbm.at[pl.ds(end0, _GB)], sw0).wait()
            pltpu.make_async_copy(rows1, out_hbm.at[pl.ds(end1, _GB)], sw1).wait()

    return _pump


_N_SLICES = 1
_SLICE = _N_EDGES // _N_SLICES
_pump_slice = _make_pump(_SLICE)


def kernel(x, vertex_id, W1, b1, W2):
    w2a = W2[:_D]
    w2b = W2[_D:]
    z = _mm1(x, W1, b1.reshape(1, _D))
    zmax_flat = _segmax_kernel(z.reshape(-1), vertex_id)
    y = _mm_small(zmax_flat.reshape(_N_NODES, _D), w2b)
    outs = []
    for s in range(_N_SLICES):
        lo = s * _SLICE
        y_exp = _pump_slice(vertex_id[lo:lo + _SLICE], y)
        outs.append(_mm2_add(x[lo:lo + _SLICE], w2a, y_exp))
    if _N_SLICES == 1:
        return outs[0]
    return jnp.concatenate(outs, axis=0)


# speedup vs baseline: 1.2056x; 1.0380x over previous
import functools

import jax
import jax.numpy as jnp
from jax import lax
from jax.experimental import pallas as pl
from jax.experimental.pallas import tpu as pltpu
from jax.experimental.pallas import tpu_sc as plsc

_N_EDGES = 320000
_N_NODES = 10000
_D = 128

_NC = 2
_NS = 16
_NW = _NC * _NS

_EBLK = 2560


def _mm1_body(x_ref, w_ref, b_ref, z_ref):
    acc = jnp.dot(x_ref[...], w_ref[...], preferred_element_type=jnp.float32)
    z_ref[...] = acc + b_ref[...]


def _mm1(x, w1, b):
    return pl.pallas_call(
        _mm1_body,
        grid=(_N_EDGES // _EBLK,),
        in_specs=[
            pl.BlockSpec((_EBLK, _D), lambda i: (i, 0)),
            pl.BlockSpec((_D, _D), lambda i: (0, 0)),
            pl.BlockSpec((1, _D), lambda i: (0, 0)),
        ],
        out_specs=pl.BlockSpec((_EBLK, _D), lambda i: (i, 0)),
        out_shape=jax.ShapeDtypeStruct((_N_EDGES, _D), jnp.float32),
    )(x, w1, b)


_EBLK2 = 2000


def _mm2_add_body(x_ref, w_ref, ye_ref, o_ref):
    acc = jnp.dot(x_ref[...], w_ref[...], preferred_element_type=jnp.float32)
    o_ref[...] = acc + ye_ref[...]


def _mm2_add(x, w2a, y_exp):
    n = x.shape[0]
    return pl.pallas_call(
        _mm2_add_body,
        grid=(n // _EBLK2,),
        in_specs=[
            pl.BlockSpec((_EBLK2, _D), lambda i: (i, 0)),
            pl.BlockSpec((_D, _D), lambda i: (0, 0)),
            pl.BlockSpec((_EBLK2, _D), lambda i: (i, 0)),
        ],
        out_specs=pl.BlockSpec((_EBLK2, _D), lambda i: (i, 0)),
        out_shape=jax.ShapeDtypeStruct((n, _D), jnp.float32),
    )(x, w2a, y_exp)


_NBLK = 2000


def _mm_small_body(zm_ref, w_ref, y_ref):
    y_ref[...] = jnp.dot(zm_ref[...], w_ref[...], preferred_element_type=jnp.float32)


def _mm_small(zmax, w2b):
    return pl.pallas_call(
        _mm_small_body,
        grid=(_N_NODES // _NBLK,),
        in_specs=[
            pl.BlockSpec((_NBLK, _D), lambda i: (i, 0)),
            pl.BlockSpec((_D, _D), lambda i: (0, 0)),
        ],
        out_specs=pl.BlockSpec((_NBLK, _D), lambda i: (i, 0)),
        out_shape=jax.ShapeDtypeStruct((_N_NODES, _D), jnp.float32),
    )(zmax, w2b)


_CHUNK = _N_EDGES // _NW
_SB = 400
_NREG = _D // 16

_sc_mesh = plsc.VectorSubcoreMesh(core_axis_name="c", subcore_axis_name="s")


_NEG = float("-inf")


@functools.partial(
    pl.kernel,
    out_type=jax.ShapeDtypeStruct((_N_NODES * _D,), jnp.float32),
    mesh=_sc_mesh,
    scratch_types=[
        pltpu.VMEM((_SB,), jnp.int32),
        pltpu.VMEM((_SB * _D,), jnp.float32),
        pltpu.VMEM((16,), jnp.int32),
        pltpu.VMEM((_D,), jnp.float32),
        pltpu.VMEM((_D,), jnp.float32),
        pltpu.VMEM((16,), jnp.int32),
        pltpu.VMEM((16,), jnp.int32),
        pltpu.VMEM((16,), jnp.int32),
        pltpu.VMEM((16,), jnp.int32),
        pltpu.VMEM((_D,), jnp.float32),
        pltpu.SemaphoreType.DMA,
        pltpu.SemaphoreType.DMA,
    ],
)
def _segmax_kernel(z_hbm, vid_hbm, zmax_hbm, ids_v, z_v, prev_v, row0, row1,
                   own_v, cur_v, prevg_v, fcnt_v, acc_v, sf0, sf1):
    wid = lax.axis_index("s") * _NC + lax.axis_index("c")
    start = wid * _CHUNK
    chunk_end = start + _CHUNK
    nblk = (_N_EDGES - start) // _SB

    @pl.when(wid > 0)
    def _():
        pltpu.sync_copy(vid_hbm.at[pl.ds(start - 16, 16)], prev_v)

    prev0 = jnp.where(wid > 0, prev_v[...][15], -1)

    def flush_slot(row_v, sem, fcnt, cur, a):
        @pl.when(fcnt >= 2)
        def _():
            pltpu.make_async_copy(row_v, zmax_hbm.at[pl.ds(0, _D)], sem).wait()
        for k in range(_NREG):
            row_v[pl.ds(16 * k, 16)] = a[k]
        pltpu.async_copy(row_v, zmax_hbm.at[pl.ds(cur * _D, _D)], sem)

    def one_edge(ge, base, eid, c):
        own, cur, prev, fcnt = c[0], c[1], c[2], c[3]
        a = c[4:]
        row = tuple(z_v[pl.ds(base + 16 * k, 16)] for k in range(_NREG))
        in_chunk = ge < chunk_end
        b = eid != prev
        flush_now = b & (own == 1)

        @pl.when(flush_now)
        def _():
            @pl.when(lax.rem(fcnt, 2) == 0)
            def _():
                flush_slot(row0, sf0, fcnt, cur, a)

            @pl.when(lax.rem(fcnt, 2) == 1)
            def _():
                flush_slot(row1, sf1, fcnt, cur, a)

        new_own = jnp.where(
            b, jnp.where(in_chunk, 1, 0), own
        )
        new_cur = jnp.where(b, eid, cur)
        new_fcnt = fcnt + jnp.where(flush_now, 1, 0)
        new_a = tuple(
            jnp.maximum(jnp.where(b, _NEG, a[k]), row[k])
            for k in range(_NREG)
        )
        return (new_own, new_cur, eid, new_fcnt) + new_a

    own_v[...] = jnp.full((16,), 0, jnp.int32)
    cur_v[...] = jnp.full((16,), -1, jnp.int32)
    prevg_v[...] = jnp.full((16,), prev0, jnp.int32)
    fcnt_v[...] = jnp.full((16,), 0, jnp.int32)

    def wbody(g, carry):
        own0 = own_v[...][0]
        blk = start + g * _SB

        @pl.when((blk < chunk_end) | (own0 == 1))
        def _():
            pltpu.sync_copy(vid_hbm.at[pl.ds(blk, _SB)], ids_v)
            pltpu.sync_copy(z_hbm.at[pl.ds(blk * _D, _SB * _D)], z_v)
            cur0 = cur_v[...][0]
            prevg0 = prevg_v[...][0]
            fcnt0 = fcnt_v[...][0]
            a0 = tuple(acc_v[pl.ds(16 * k, 16)] for k in range(_NREG))

            def group_step(gi, c):
                idvec = ids_v[pl.ds(gi * 16, 16)]
                for l in range(16):
                    c = one_edge(
                        blk + gi * 16 + l, (gi * 16 + l) * _D, idvec[l], c
                    )
                return c

            res = lax.fori_loop(
                0, _SB // 16, group_step, (own0, cur0, prevg0, fcnt0) + a0
            )
            own_v[...] = jnp.full((16,), res[0], jnp.int32)
            cur_v[...] = jnp.full((16,), res[1], jnp.int32)
            prevg_v[...] = jnp.full((16,), res[2], jnp.int32)
            fcnt_v[...] = jnp.full((16,), res[3], jnp.int32)
            for k in range(_NREG):
                acc_v[pl.ds(16 * k, 16)] = res[4 + k]

        return carry

    lax.fori_loop(0, nblk, wbody, 0)

    own = own_v[...][0]
    cur = cur_v[...][0]

    @pl.when(own == 1)
    def _():
        pltpu.sync_copy(acc_v, zmax_hbm.at[pl.ds(cur * _D, _D)])

    fcnt = fcnt_v[...][0]

    @pl.when(fcnt >= 1)
    def _():
        last = lax.rem(fcnt - 1, 2)

        @pl.when(last == 0)
        def _():
            pltpu.make_async_copy(row0, zmax_hbm.at[pl.ds(0, _D)], sf0).wait()

        @pl.when(last == 1)
        def _():
            pltpu.make_async_copy(row1, zmax_hbm.at[pl.ds(0, _D)], sf1).wait()

    @pl.when(fcnt >= 2)
    def _():
        last2 = lax.rem(fcnt - 2, 2)

        @pl.when(last2 == 0)
        def _():
            pltpu.make_async_copy(row0, zmax_hbm.at[pl.ds(0, _D)], sf0).wait()

        @pl.when(last2 == 1)
        def _():
            pltpu.make_async_copy(row1, zmax_hbm.at[pl.ds(0, _D)], sf1).wait()


_GB = 400


def _make_pump(n_slice):
    chunk = n_slice // _NW
    tpw = chunk // _GB
    pairs = tpw // 2

    @functools.partial(
        pl.kernel,
        out_type=jax.ShapeDtypeStruct((n_slice, _D), jnp.float32),
        mesh=_sc_mesh,
        scratch_types=[
            pltpu.VMEM((_GB,), jnp.int32),
            pltpu.VMEM((_GB,), jnp.int32),
            pltpu.VMEM((_GB, _D), jnp.float32),
            pltpu.VMEM((_GB, _D), jnp.float32),
            pltpu.SemaphoreType.DMA,
            pltpu.SemaphoreType.DMA,
            pltpu.SemaphoreType.DMA,
            pltpu.SemaphoreType.DMA,
        ],
    )
    def _pump(vid_hbm, y_hbm, out_hbm, idx0, idx1, rows0, rows1,
              sg0, sg1, sw0, sw1):
        wid = lax.axis_index("s") * _NC + lax.axis_index("c")
        start = wid * chunk

        def pair(i, carry):
            b0 = start + (2 * i) * _GB
            b1 = start + (2 * i + 1) * _GB

            @pl.when(i > 0)
            def _():
                pltpu.make_async_copy(
                    rows0, out_hbm.at[pl.ds(b0, _GB)], sw0).wait()
                pltpu.make_async_copy(
                    rows1, out_hbm.at[pl.ds(b1, _GB)], sw1).wait()

            pltpu.sync_copy(vid_hbm.at[pl.ds(b0, _GB)], idx0)
            pltpu.async_copy(y_hbm.at[idx0], rows0, sg0)
            pltpu.sync_copy(vid_hbm.at[pl.ds(b1, _GB)], idx1)
            pltpu.async_copy(y_hbm.at[idx1], rows1, sg1)

            pltpu.make_async_copy(y_hbm.at[idx0], rows0, sg0).wait()
            pltpu.async_copy(rows0, out_hbm.at[pl.ds(b0, _GB)], sw0)
            pltpu.make_async_copy(y_hbm.at[idx1], rows1, sg1).wait()
            pltpu.async_copy(rows1, out_hbm.at[pl.ds(b1, _GB)], sw1)
            return carry

        lax.fori_loop(0, pairs, pair, 0)
        if tpw % 2 == 1:
            bt = start + (tpw - 1) * _GB
            if pairs > 0:
                pltpu.make_async_copy(
                    rows0, out_hbm.at[pl.ds(bt - 2 * _GB, _GB)], sw0).wait()
            pltpu.sync_copy(vid_hbm.at[pl.ds(bt, _GB)], idx0)
            pltpu.async_copy(y_hbm.at[idx0], rows0, sg0)
            pltpu.make_async_copy(y_hbm.at[idx0], rows0, sg0).wait()
            pltpu.async_copy(rows0, out_hbm.at[pl.ds(bt, _GB)], sw0)
            if pairs > 0:
                pltpu.make_async_copy(
                    rows1, out_hbm.at[pl.ds(bt - _GB, _GB)], sw1).wait()
            pltpu.make_async_copy(rows0, out_hbm.at[pl.ds(bt, _GB)], sw0).wait()
        else:
            end0 = start + (tpw - 2) * _GB
            end1 = start + (tpw - 1) * _GB
            pltpu.make_async_copy(rows0, out_hbm.at[pl.ds(end0, _GB)], sw0).wait()
            pltpu.make_async_copy(rows1, out_hbm.at[pl.ds(end1, _GB)], sw1).wait()

    return _pump


_N_SLICES = 1
_SLICE = _N_EDGES // _N_SLICES
_pump_slice = _make_pump(_SLICE)


def kernel(x, vertex_id, W1, b1, W2):
    w2a = W2[:_D]
    w2b = W2[_D:]
    z = _mm1(x, W1, b1.reshape(1, _D))
    zmax_flat = _segmax_kernel(z.reshape(-1), vertex_id)
    y = _mm_small(zmax_flat.reshape(_N_NODES, _D), w2b)
    outs = []
    for s in range(_N_SLICES):
        lo = s * _SLICE
        y_exp = _pump_slice(vertex_id[lo:lo + _SLICE], y)
        outs.append(_mm2_add(x[lo:lo + _SLICE], w2a, y_exp))
    if _N_SLICES == 1:
        return outs[0]
    return jnp.concatenate(outs, axis=0)


# speedup vs baseline: 1.3057x; 1.0830x over previous
import functools

import jax
import jax.numpy as jnp
from jax import lax
from jax.experimental import pallas as pl
from jax.experimental.pallas import tpu as pltpu
from jax.experimental.pallas import tpu_sc as plsc

_N_EDGES = 320000
_N_NODES = 10000
_D = 128

_NC = 2
_NS = 16
_NW = _NC * _NS

_EBLK = 2560


def _mm1_body(x_ref, w_ref, b_ref, z_ref):
    acc = jnp.dot(x_ref[...], w_ref[...], preferred_element_type=jnp.float32)
    z_ref[...] = acc + b_ref[...]


def _mm1(x, w1, b):
    return pl.pallas_call(
        _mm1_body,
        grid=(_N_EDGES // _EBLK,),
        in_specs=[
            pl.BlockSpec((_EBLK, _D), lambda i: (i, 0)),
            pl.BlockSpec((_D, _D), lambda i: (0, 0)),
            pl.BlockSpec((1, _D), lambda i: (0, 0)),
        ],
        out_specs=pl.BlockSpec((_EBLK, _D), lambda i: (i, 0)),
        out_shape=jax.ShapeDtypeStruct((_N_EDGES, _D), jnp.float32),
    )(x, w1, b)


_EBLK2 = 2000


def _mm2_add_body(x_ref, w_ref, ye_ref, o_ref):
    acc = jnp.dot(x_ref[...], w_ref[...], preferred_element_type=jnp.float32)
    o_ref[...] = acc + ye_ref[...]


def _mm2_add(x, w2a, y_exp):
    n = x.shape[0]
    return pl.pallas_call(
        _mm2_add_body,
        grid=(n // _EBLK2,),
        in_specs=[
            pl.BlockSpec((_EBLK2, _D), lambda i: (i, 0)),
            pl.BlockSpec((_D, _D), lambda i: (0, 0)),
            pl.BlockSpec((_EBLK2, _D), lambda i: (i, 0)),
        ],
        out_specs=pl.BlockSpec((_EBLK2, _D), lambda i: (i, 0)),
        out_shape=jax.ShapeDtypeStruct((n, _D), jnp.float32),
    )(x, w2a, y_exp)


_NBLK = 2000


def _mm_small_body(zm_ref, w_ref, y_ref):
    y_ref[...] = jnp.dot(zm_ref[...], w_ref[...], preferred_element_type=jnp.float32)


def _mm_small(zmax, w2b):
    return pl.pallas_call(
        _mm_small_body,
        grid=(_N_NODES // _NBLK,),
        in_specs=[
            pl.BlockSpec((_NBLK, _D), lambda i: (i, 0)),
            pl.BlockSpec((_D, _D), lambda i: (0, 0)),
        ],
        out_specs=pl.BlockSpec((_NBLK, _D), lambda i: (i, 0)),
        out_shape=jax.ShapeDtypeStruct((_N_NODES, _D), jnp.float32),
    )(zmax, w2b)


_CHUNK = _N_EDGES // _NW
_SB = 400
_NREG = _D // 16

_sc_mesh = plsc.VectorSubcoreMesh(core_axis_name="c", subcore_axis_name="s")


_NEG = float("-inf")


@functools.partial(
    pl.kernel,
    out_type=jax.ShapeDtypeStruct((_N_NODES * _D,), jnp.float32),
    mesh=_sc_mesh,
    scratch_types=[
        pltpu.VMEM((_SB,), jnp.int32),
        pltpu.VMEM((_SB,), jnp.int32),
        pltpu.VMEM((_SB * _D,), jnp.float32),
        pltpu.VMEM((_SB * _D,), jnp.float32),
        pltpu.VMEM((16,), jnp.int32),
        pltpu.VMEM((_D,), jnp.float32),
        pltpu.VMEM((_D,), jnp.float32),
        pltpu.VMEM((16,), jnp.int32),
        pltpu.VMEM((16,), jnp.int32),
        pltpu.VMEM((16,), jnp.int32),
        pltpu.VMEM((16,), jnp.int32),
        pltpu.VMEM((_D,), jnp.float32),
        pltpu.SemaphoreType.DMA,
        pltpu.SemaphoreType.DMA,
        pltpu.SemaphoreType.DMA,
        pltpu.SemaphoreType.DMA,
        pltpu.SemaphoreType.DMA,
        pltpu.SemaphoreType.DMA,
    ],
)
def _segmax_kernel(z_hbm, vid_hbm, zmax_hbm, ids0_v, ids1_v, z0_v, z1_v,
                   prev_v, row0, row1, own_v, cur_v, prevg_v, fcnt_v, acc_v,
                   si0, si1, sz0, sz1, sf0, sf1):
    wid = lax.axis_index("s") * _NC + lax.axis_index("c")
    start = wid * _CHUNK
    chunk_end = start + _CHUNK
    _P1 = _CHUNK // _SB
    _PAIRS = _P1 // 2

    @pl.when(wid > 0)
    def _():
        pltpu.sync_copy(vid_hbm.at[pl.ds(start - 16, 16)], prev_v)

    prev0 = jnp.where(wid > 0, prev_v[...][15], -1)

    def flush_slot(row_v, sem, fcnt, cur, a):
        @pl.when(fcnt >= 2)
        def _():
            pltpu.make_async_copy(row_v, zmax_hbm.at[pl.ds(0, _D)], sem).wait()
        for k in range(_NREG):
            row_v[pl.ds(16 * k, 16)] = a[k]
        pltpu.async_copy(row_v, zmax_hbm.at[pl.ds(cur * _D, _D)], sem)

    def one_edge(z_ref, ge, base, eid, c):
        own, cur, prev, fcnt = c[0], c[1], c[2], c[3]
        a = c[4:]
        row = tuple(z_ref[pl.ds(base + 16 * k, 16)] for k in range(_NREG))
        in_chunk = ge < chunk_end
        b = eid != prev
        flush_now = b & (own == 1)

        @pl.when(flush_now)
        def _():
            @pl.when(lax.rem(fcnt, 2) == 0)
            def _():
                flush_slot(row0, sf0, fcnt, cur, a)

            @pl.when(lax.rem(fcnt, 2) == 1)
            def _():
                flush_slot(row1, sf1, fcnt, cur, a)

        new_own = jnp.where(
            b, jnp.where(in_chunk, 1, 0), own
        )
        new_cur = jnp.where(b, eid, cur)
        new_fcnt = fcnt + jnp.where(flush_now, 1, 0)
        new_a = tuple(
            jnp.maximum(jnp.where(b, _NEG, a[k]), row[k])
            for k in range(_NREG)
        )
        return (new_own, new_cur, eid, new_fcnt) + new_a

    own_v[...] = jnp.full((16,), 0, jnp.int32)
    cur_v[...] = jnp.full((16,), -1, jnp.int32)
    prevg_v[...] = jnp.full((16,), prev0, jnp.int32)
    fcnt_v[...] = jnp.full((16,), 0, jnp.int32)

    def scan_block(ids_ref, z_ref, blk):
        own0 = own_v[...][0]
        cur0 = cur_v[...][0]
        prevg0 = prevg_v[...][0]
        fcnt0 = fcnt_v[...][0]
        a0 = tuple(acc_v[pl.ds(16 * k, 16)] for k in range(_NREG))

        def group_step(gi, c):
            idvec = ids_ref[pl.ds(gi * 16, 16)]
            for l in range(16):
                c = one_edge(
                    z_ref, blk + gi * 16 + l, (gi * 16 + l) * _D, idvec[l], c
                )
            return c

        res = lax.fori_loop(
            0, _SB // 16, group_step, (own0, cur0, prevg0, fcnt0) + a0
        )
        own_v[...] = jnp.full((16,), res[0], jnp.int32)
        cur_v[...] = jnp.full((16,), res[1], jnp.int32)
        prevg_v[...] = jnp.full((16,), res[2], jnp.int32)
        fcnt_v[...] = jnp.full((16,), res[3], jnp.int32)
        for k in range(_NREG):
            acc_v[pl.ds(16 * k, 16)] = res[4 + k]

    def fetch(blk, ids_ref, z_ref, sem_i, sem_z):
        pltpu.async_copy(vid_hbm.at[pl.ds(blk, _SB)], ids_ref, sem_i)
        pltpu.async_copy(z_hbm.at[pl.ds(blk * _D, _SB * _D)], z_ref, sem_z)

    def fetch_wait(blk, ids_ref, z_ref, sem_i, sem_z):
        pltpu.make_async_copy(
            vid_hbm.at[pl.ds(blk, _SB)], ids_ref, sem_i).wait()
        pltpu.make_async_copy(
            z_hbm.at[pl.ds(blk * _D, _SB * _D)], z_ref, sem_z).wait()

    fetch(start, ids0_v, z0_v, si0, sz0)
    fetch(start + _SB, ids1_v, z1_v, si1, sz1)

    def pair_body(i, carry):
        b0 = start + (2 * i) * _SB
        b1 = b0 + _SB
        fetch_wait(b0, ids0_v, z0_v, si0, sz0)
        scan_block(ids0_v, z0_v, b0)

        @pl.when(2 * i + 2 < _P1)
        def _():
            fetch(b0 + 2 * _SB, ids0_v, z0_v, si0, sz0)

        fetch_wait(b1, ids1_v, z1_v, si1, sz1)
        scan_block(ids1_v, z1_v, b1)

        @pl.when(2 * i + 3 < _P1)
        def _():
            fetch(b1 + 2 * _SB, ids1_v, z1_v, si1, sz1)

        return carry

    lax.fori_loop(0, _PAIRS, pair_body, 0)
    if _P1 % 2 == 1:
        bt = start + (_P1 - 1) * _SB
        fetch_wait(bt, ids0_v, z0_v, si0, sz0)
        scan_block(ids0_v, z0_v, bt)

    nspill = (_N_EDGES - chunk_end) // _SB

    def spill_body(g, carry):
        blk = chunk_end + g * _SB

        @pl.when(own_v[...][0] == 1)
        def _():
            pltpu.sync_copy(vid_hbm.at[pl.ds(blk, _SB)], ids0_v)
            pltpu.sync_copy(z_hbm.at[pl.ds(blk * _D, _SB * _D)], z0_v)
            scan_block(ids0_v, z0_v, blk)

        return carry

    lax.fori_loop(0, nspill, spill_body, 0)

    own = own_v[...][0]
    cur = cur_v[...][0]

    @pl.when(own == 1)
    def _():
        pltpu.sync_copy(acc_v, zmax_hbm.at[pl.ds(cur * _D, _D)])

    fcnt = fcnt_v[...][0]

    @pl.when(fcnt >= 1)
    def _():
        last = lax.rem(fcnt - 1, 2)

        @pl.when(last == 0)
        def _():
            pltpu.make_async_copy(row0, zmax_hbm.at[pl.ds(0, _D)], sf0).wait()

        @pl.when(last == 1)
        def _():
            pltpu.make_async_copy(row1, zmax_hbm.at[pl.ds(0, _D)], sf1).wait()

    @pl.when(fcnt >= 2)
    def _():
        last2 = lax.rem(fcnt - 2, 2)

        @pl.when(last2 == 0)
        def _():
            pltpu.make_async_copy(row0, zmax_hbm.at[pl.ds(0, _D)], sf0).wait()

        @pl.when(last2 == 1)
        def _():
            pltpu.make_async_copy(row1, zmax_hbm.at[pl.ds(0, _D)], sf1).wait()


_GB = 400


def _make_pump(n_slice):
    chunk = n_slice // _NW
    tpw = chunk // _GB
    pairs = tpw // 2

    @functools.partial(
        pl.kernel,
        out_type=jax.ShapeDtypeStruct((n_slice, _D), jnp.float32),
        mesh=_sc_mesh,
        scratch_types=[
            pltpu.VMEM((_GB,), jnp.int32),
            pltpu.VMEM((_GB,), jnp.int32),
            pltpu.VMEM((_GB, _D), jnp.float32),
            pltpu.VMEM((_GB, _D), jnp.float32),
            pltpu.SemaphoreType.DMA,
            pltpu.SemaphoreType.DMA,
            pltpu.SemaphoreType.DMA,
            pltpu.SemaphoreType.DMA,
        ],
    )
    def _pump(vid_hbm, y_hbm, out_hbm, idx0, idx1, rows0, rows1,
              sg0, sg1, sw0, sw1):
        wid = lax.axis_index("s") * _NC + lax.axis_index("c")
        start = wid * chunk

        def pair(i, carry):
            b0 = start + (2 * i) * _GB
            b1 = start + (2 * i + 1) * _GB

            @pl.when(i > 0)
            def _():
                pltpu.make_async_copy(
                    rows0, out_hbm.at[pl.ds(b0, _GB)], sw0).wait()
                pltpu.make_async_copy(
                    rows1, out_hbm.at[pl.ds(b1, _GB)], sw1).wait()

            pltpu.sync_copy(vid_hbm.at[pl.ds(b0, _GB)], idx0)
            pltpu.async_copy(y_hbm.at[idx0], rows0, sg0)
            pltpu.sync_copy(vid_hbm.at[pl.ds(b1, _GB)], idx1)
            pltpu.async_copy(y_hbm.at[idx1], rows1, sg1)

            pltpu.make_async_copy(y_hbm.at[idx0], rows0, sg0).wait()
            pltpu.async_copy(rows0, out_hbm.at[pl.ds(b0, _GB)], sw0)
            pltpu.make_async_copy(y_hbm.at[idx1], rows1, sg1).wait()
            pltpu.async_copy(rows1, out_hbm.at[pl.ds(b1, _GB)], sw1)
            return carry

        lax.fori_loop(0, pairs, pair, 0)
        if tpw % 2 == 1:
            bt = start + (tpw - 1) * _GB
            if pairs > 0:
                pltpu.make_async_copy(
                    rows0, out_hbm.at[pl.ds(bt - 2 * _GB, _GB)], sw0).wait()
            pltpu.sync_copy(vid_hbm.at[pl.ds(bt, _GB)], idx0)
            pltpu.async_copy(y_hbm.at[idx0], rows0, sg0)
            pltpu.make_async_copy(y_hbm.at[idx0], rows0, sg0).wait()
            pltpu.async_copy(rows0, out_hbm.at[pl.ds(bt, _GB)], sw0)
            if pairs > 0:
                pltpu.make_async_copy(
                    rows1, out_hbm.at[pl.ds(bt - _GB, _GB)], sw1).wait()
            pltpu.make_async_copy(rows0, out_hbm.at[pl.ds(bt, _GB)], sw0).wait()
        else:
            end0 = start + (tpw - 2) * _GB
            end1 = start + (tpw - 1) * _GB
            pltpu.make_async_copy(rows0, out_hbm.at[pl.ds(end0, _GB)], sw0).wait()
            pltpu.make_async_copy(rows1, out_hbm.at[pl.ds(end1, _GB)], sw1).wait()

    return _pump


_N_SLICES = 1
_SLICE = _N_EDGES // _N_SLICES
_pump_slice = _make_pump(_SLICE)


def kernel(x, vertex_id, W1, b1, W2):
    w2a = W2[:_D]
    w2b = W2[_D:]
    z = _mm1(x, W1, b1.reshape(1, _D))
    zmax_flat = _segmax_kernel(z.reshape(-1), vertex_id)
    y = _mm_small(zmax_flat.reshape(_N_NODES, _D), w2b)
    outs = []
    for s in range(_N_SLICES):
        lo = s * _SLICE
        y_exp = _pump_slice(vertex_id[lo:lo + _SLICE], y)
        outs.append(_mm2_add(x[lo:lo + _SLICE], w2a, y_exp))
    if _N_SLICES == 1:
        return outs[0]
    return jnp.concatenate(outs, axis=0)


# speedup vs baseline: 1.4545x; 1.1140x over previous
import functools

import jax
import jax.numpy as jnp
from jax import lax
from jax.experimental import pallas as pl
from jax.experimental.pallas import tpu as pltpu
from jax.experimental.pallas import tpu_sc as plsc

_N_EDGES = 320000
_N_NODES = 10000
_D = 128

_NC = 2
_NS = 16
_NW = _NC * _NS

_EBLK = 6400


def _mm1_body(x_ref, w_ref, b_ref, z_ref):
    acc = jnp.dot(x_ref[...], w_ref[...], preferred_element_type=jnp.float32)
    z_ref[...] = acc + b_ref[...]


def _mm1(x, w1, b):
    return pl.pallas_call(
        _mm1_body,
        grid=(_N_EDGES // _EBLK,),
        in_specs=[
            pl.BlockSpec((_EBLK, _D), lambda i: (i, 0)),
            pl.BlockSpec((_D, _D), lambda i: (0, 0)),
            pl.BlockSpec((1, _D), lambda i: (0, 0)),
        ],
        out_specs=pl.BlockSpec((_EBLK, _D), lambda i: (i, 0)),
        out_shape=jax.ShapeDtypeStruct((_N_EDGES, _D), jnp.float32),
    )(x, w1, b)


_EBLK2 = 4000


def _mm2_add_body(x_ref, w_ref, ye_ref, o_ref):
    acc = jnp.dot(x_ref[...], w_ref[...], preferred_element_type=jnp.float32)
    o_ref[...] = acc + ye_ref[...]


def _mm2_add(x, w2a, y_exp):
    n = x.shape[0]
    return pl.pallas_call(
        _mm2_add_body,
        grid=(n // _EBLK2,),
        in_specs=[
            pl.BlockSpec((_EBLK2, _D), lambda i: (i, 0)),
            pl.BlockSpec((_D, _D), lambda i: (0, 0)),
            pl.BlockSpec((_EBLK2, _D), lambda i: (i, 0)),
        ],
        out_specs=pl.BlockSpec((_EBLK2, _D), lambda i: (i, 0)),
        out_shape=jax.ShapeDtypeStruct((n, _D), jnp.float32),
    )(x, w2a, y_exp)


_NBLK = 2000


def _mm_small_body(zm_ref, w_ref, y_ref):
    y_ref[...] = jnp.dot(zm_ref[...], w_ref[...], preferred_element_type=jnp.float32)


def _mm_small(zmax, w2b):
    return pl.pallas_call(
        _mm_small_body,
        grid=(_N_NODES // _NBLK,),
        in_specs=[
            pl.BlockSpec((_NBLK, _D), lambda i: (i, 0)),
            pl.BlockSpec((_D, _D), lambda i: (0, 0)),
        ],
        out_specs=pl.BlockSpec((_NBLK, _D), lambda i: (i, 0)),
        out_shape=jax.ShapeDtypeStruct((_N_NODES, _D), jnp.float32),
    )(zmax, w2b)


_CHUNK = _N_EDGES // _NW
_SB = 400
_NREG = _D // 16

_sc_mesh = plsc.VectorSubcoreMesh(core_axis_name="c", subcore_axis_name="s")


_NEG = float("-inf")


@functools.partial(
    pl.kernel,
    out_type=jax.ShapeDtypeStruct((_N_NODES * _D,), jnp.float32),
    mesh=_sc_mesh,
    scratch_types=[
        pltpu.VMEM((_SB,), jnp.int32),
        pltpu.VMEM((_SB,), jnp.int32),
        pltpu.VMEM((_SB * _D,), jnp.float32),
        pltpu.VMEM((_SB * _D,), jnp.float32),
        pltpu.VMEM((16,), jnp.int32),
        pltpu.VMEM((_D,), jnp.float32),
        pltpu.VMEM((_D,), jnp.float32),
        pltpu.VMEM((16,), jnp.int32),
        pltpu.VMEM((16,), jnp.int32),
        pltpu.VMEM((16,), jnp.int32),
        pltpu.VMEM((16,), jnp.int32),
        pltpu.VMEM((_D,), jnp.float32),
        pltpu.SemaphoreType.DMA,
        pltpu.SemaphoreType.DMA,
        pltpu.SemaphoreType.DMA,
        pltpu.SemaphoreType.DMA,
        pltpu.SemaphoreType.DMA,
        pltpu.SemaphoreType.DMA,
    ],
)
def _segmax_kernel(z_hbm, vid_hbm, zmax_hbm, ids0_v, ids1_v, z0_v, z1_v,
                   prev_v, row0, row1, own_v, cur_v, prevg_v, fcnt_v, acc_v,
                   si0, si1, sz0, sz1, sf0, sf1):
    wid = lax.axis_index("s") * _NC + lax.axis_index("c")
    start = wid * _CHUNK
    chunk_end = start + _CHUNK
    _P1 = _CHUNK // _SB
    _PAIRS = _P1 // 2

    @pl.when(wid > 0)
    def _():
        pltpu.sync_copy(vid_hbm.at[pl.ds(start - 16, 16)], prev_v)

    prev0 = jnp.where(wid > 0, prev_v[...][15], -1)

    def flush_slot(row_v, sem, fcnt, cur, a):
        @pl.when(fcnt >= 2)
        def _():
            pltpu.make_async_copy(row_v, zmax_hbm.at[pl.ds(0, _D)], sem).wait()
        for k in range(_NREG):
            row_v[pl.ds(16 * k, 16)] = a[k]
        pltpu.async_copy(row_v, zmax_hbm.at[pl.ds(cur * _D, _D)], sem)

    def one_edge(z_ref, ge, base, eid, c):
        own, cur, prev, fcnt = c[0], c[1], c[2], c[3]
        a = c[4:]
        row = tuple(z_ref[pl.ds(base + 16 * k, 16)] for k in range(_NREG))
        in_chunk = ge < chunk_end
        b = eid != prev
        flush_now = b & (own == 1)

        @pl.when(flush_now)
        def _():
            @pl.when(lax.rem(fcnt, 2) == 0)
            def _():
                flush_slot(row0, sf0, fcnt, cur, a)

            @pl.when(lax.rem(fcnt, 2) == 1)
            def _():
                flush_slot(row1, sf1, fcnt, cur, a)

        new_own = jnp.where(
            b, jnp.where(in_chunk, 1, 0), own
        )
        new_cur = jnp.where(b, eid, cur)
        new_fcnt = fcnt + jnp.where(flush_now, 1, 0)
        new_a = tuple(
            jnp.maximum(jnp.where(b, _NEG, a[k]), row[k])
            for k in range(_NREG)
        )
        return (new_own, new_cur, eid, new_fcnt) + new_a

    own_v[...] = jnp.full((16,), 0, jnp.int32)
    cur_v[...] = jnp.full((16,), -1, jnp.int32)
    prevg_v[...] = jnp.full((16,), prev0, jnp.int32)
    fcnt_v[...] = jnp.full((16,), 0, jnp.int32)

    def scan_block(ids_ref, z_ref, blk):
        own0 = own_v[...][0]
        cur0 = cur_v[...][0]
        prevg0 = prevg_v[...][0]
        fcnt0 = fcnt_v[...][0]
        a0 = tuple(acc_v[pl.ds(16 * k, 16)] for k in range(_NREG))

        def group_step(gi, c):
            idvec = ids_ref[pl.ds(gi * 16, 16)]
            for l in range(16):
                c = one_edge(
                    z_ref, blk + gi * 16 + l, (gi * 16 + l) * _D, idvec[l], c
                )
            return c

        res = lax.fori_loop(
            0, _SB // 16, group_step, (own0, cur0, prevg0, fcnt0) + a0
        )
        own_v[...] = jnp.full((16,), res[0], jnp.int32)
        cur_v[...] = jnp.full((16,), res[1], jnp.int32)
        prevg_v[...] = jnp.full((16,), res[2], jnp.int32)
        fcnt_v[...] = jnp.full((16,), res[3], jnp.int32)
        for k in range(_NREG):
            acc_v[pl.ds(16 * k, 16)] = res[4 + k]

    def fetch(blk, ids_ref, z_ref, sem_i, sem_z):
        pltpu.async_copy(vid_hbm.at[pl.ds(blk, _SB)], ids_ref, sem_i)
        pltpu.async_copy(z_hbm.at[pl.ds(blk * _D, _SB * _D)], z_ref, sem_z)

    def fetch_wait(blk, ids_ref, z_ref, sem_i, sem_z):
        pltpu.make_async_copy(
            vid_hbm.at[pl.ds(blk, _SB)], ids_ref, sem_i).wait()
        pltpu.make_async_copy(
            z_hbm.at[pl.ds(blk * _D, _SB * _D)], z_ref, sem_z).wait()

    fetch(start, ids0_v, z0_v, si0, sz0)
    fetch(start + _SB, ids1_v, z1_v, si1, sz1)

    def pair_body(i, carry):
        b0 = start + (2 * i) * _SB
        b1 = b0 + _SB
        fetch_wait(b0, ids0_v, z0_v, si0, sz0)
        scan_block(ids0_v, z0_v, b0)

        @pl.when(2 * i + 2 < _P1)
        def _():
            fetch(b0 + 2 * _SB, ids0_v, z0_v, si0, sz0)

        fetch_wait(b1, ids1_v, z1_v, si1, sz1)
        scan_block(ids1_v, z1_v, b1)

        @pl.when(2 * i + 3 < _P1)
        def _():
            fetch(b1 + 2 * _SB, ids1_v, z1_v, si1, sz1)

        return carry

    lax.fori_loop(0, _PAIRS, pair_body, 0)
    if _P1 % 2 == 1:
        bt = start + (_P1 - 1) * _SB
        fetch_wait(bt, ids0_v, z0_v, si0, sz0)
        scan_block(ids0_v, z0_v, bt)

    nspill = (_N_EDGES - chunk_end) // _SB

    def spill_body(g, carry):
        blk = chunk_end + g * _SB

        @pl.when(own_v[...][0] == 1)
        def _():
            pltpu.sync_copy(vid_hbm.at[pl.ds(blk, _SB)], ids0_v)
            pltpu.sync_copy(z_hbm.at[pl.ds(blk * _D, _SB * _D)], z0_v)
            scan_block(ids0_v, z0_v, blk)

        return carry

    lax.fori_loop(0, nspill, spill_body, 0)

    own = own_v[...][0]
    cur = cur_v[...][0]

    @pl.when(own == 1)
    def _():
        pltpu.sync_copy(acc_v, zmax_hbm.at[pl.ds(cur * _D, _D)])

    fcnt = fcnt_v[...][0]

    @pl.when(fcnt >= 1)
    def _():
        last = lax.rem(fcnt - 1, 2)

        @pl.when(last == 0)
        def _():
            pltpu.make_async_copy(row0, zmax_hbm.at[pl.ds(0, _D)], sf0).wait()

        @pl.when(last == 1)
        def _():
            pltpu.make_async_copy(row1, zmax_hbm.at[pl.ds(0, _D)], sf1).wait()

    @pl.when(fcnt >= 2)
    def _():
        last2 = lax.rem(fcnt - 2, 2)

        @pl.when(last2 == 0)
        def _():
            pltpu.make_async_copy(row0, zmax_hbm.at[pl.ds(0, _D)], sf0).wait()

        @pl.when(last2 == 1)
        def _():
            pltpu.make_async_copy(row1, zmax_hbm.at[pl.ds(0, _D)], sf1).wait()


_GB = 400


def _make_pump(n_slice):
    chunk = n_slice // _NW
    tpw = chunk // _GB
    pairs = tpw // 2

    @functools.partial(
        pl.kernel,
        out_type=jax.ShapeDtypeStruct((n_slice, _D), jnp.float32),
        mesh=_sc_mesh,
        scratch_types=[
            pltpu.VMEM((_GB,), jnp.int32),
            pltpu.VMEM((_GB,), jnp.int32),
            pltpu.VMEM((_GB, _D), jnp.float32),
            pltpu.VMEM((_GB, _D), jnp.float32),
            pltpu.SemaphoreType.DMA,
            pltpu.SemaphoreType.DMA,
            pltpu.SemaphoreType.DMA,
            pltpu.SemaphoreType.DMA,
        ],
    )
    def _pump(vid_hbm, y_hbm, out_hbm, idx0, idx1, rows0, rows1,
              sg0, sg1, sw0, sw1):
        wid = lax.axis_index("s") * _NC + lax.axis_index("c")
        start = wid * chunk

        def pair(i, carry):
            b0 = start + (2 * i) * _GB
            b1 = start + (2 * i + 1) * _GB

            @pl.when(i > 0)
            def _():
                pltpu.make_async_copy(
                    rows0, out_hbm.at[pl.ds(b0, _GB)], sw0).wait()
                pltpu.make_async_copy(
                    rows1, out_hbm.at[pl.ds(b1, _GB)], sw1).wait()

            pltpu.sync_copy(vid_hbm.at[pl.ds(b0, _GB)], idx0)
            pltpu.async_copy(y_hbm.at[idx0], rows0, sg0)
            pltpu.sync_copy(vid_hbm.at[pl.ds(b1, _GB)], idx1)
            pltpu.async_copy(y_hbm.at[idx1], rows1, sg1)

            pltpu.make_async_copy(y_hbm.at[idx0], rows0, sg0).wait()
            pltpu.async_copy(rows0, out_hbm.at[pl.ds(b0, _GB)], sw0)
            pltpu.make_async_copy(y_hbm.at[idx1], rows1, sg1).wait()
            pltpu.async_copy(rows1, out_hbm.at[pl.ds(b1, _GB)], sw1)
            return carry

        lax.fori_loop(0, pairs, pair, 0)
        if tpw % 2 == 1:
            bt = start + (tpw - 1) * _GB
            if pairs > 0:
                pltpu.make_async_copy(
                    rows0, out_hbm.at[pl.ds(bt - 2 * _GB, _GB)], sw0).wait()
            pltpu.sync_copy(vid_hbm.at[pl.ds(bt, _GB)], idx0)
            pltpu.async_copy(y_hbm.at[idx0], rows0, sg0)
            pltpu.make_async_copy(y_hbm.at[idx0], rows0, sg0).wait()
            pltpu.async_copy(rows0, out_hbm.at[pl.ds(bt, _GB)], sw0)
            if pairs > 0:
                pltpu.make_async_copy(
                    rows1, out_hbm.at[pl.ds(bt - _GB, _GB)], sw1).wait()
            pltpu.make_async_copy(rows0, out_hbm.at[pl.ds(bt, _GB)], sw0).wait()
        else:
            end0 = start + (tpw - 2) * _GB
            end1 = start + (tpw - 1) * _GB
            pltpu.make_async_copy(rows0, out_hbm.at[pl.ds(end0, _GB)], sw0).wait()
            pltpu.make_async_copy(rows1, out_hbm.at[pl.ds(end1, _GB)], sw1).wait()

    return _pump


_N_SLICES = 1
_SLICE = _N_EDGES // _N_SLICES
_pump_slice = _make_pump(_SLICE)


def kernel(x, vertex_id, W1, b1, W2):
    w2a = W2[:_D]
    w2b = W2[_D:]
    z = _mm1(x, W1, b1.reshape(1, _D))
    zmax_flat = _segmax_kernel(z.reshape(-1), vertex_id)
    y = _mm_small(zmax_flat.reshape(_N_NODES, _D), w2b)
    outs = []
    for s in range(_N_SLICES):
        lo = s * _SLICE
        y_exp = _pump_slice(vertex_id[lo:lo + _SLICE], y)
        outs.append(_mm2_add(x[lo:lo + _SLICE], w2a, y_exp))
    if _N_SLICES == 1:
        return outs[0]
    return jnp.concatenate(outs, axis=0)


# speedup vs baseline: 1.4737x; 1.0132x over previous
import functools

import jax
import jax.numpy as jnp
from jax import lax
from jax.experimental import pallas as pl
from jax.experimental.pallas import tpu as pltpu
from jax.experimental.pallas import tpu_sc as plsc

_N_EDGES = 320000
_N_NODES = 10000
_D = 128

_NC = 2
_NS = 16
_NW = _NC * _NS

_EBLK = 12800


def _mm1_body(x_ref, w_ref, b_ref, z_ref):
    acc = jnp.dot(x_ref[...], w_ref[...], preferred_element_type=jnp.float32)
    z_ref[...] = acc + b_ref[...]


def _mm1(x, w1, b):
    return pl.pallas_call(
        _mm1_body,
        grid=(_N_EDGES // _EBLK,),
        in_specs=[
            pl.BlockSpec((_EBLK, _D), lambda i: (i, 0)),
            pl.BlockSpec((_D, _D), lambda i: (0, 0)),
            pl.BlockSpec((1, _D), lambda i: (0, 0)),
        ],
        out_specs=pl.BlockSpec((_EBLK, _D), lambda i: (i, 0)),
        out_shape=jax.ShapeDtypeStruct((_N_EDGES, _D), jnp.float32),
    )(x, w1, b)


_EBLK2 = 8000


def _mm2_add_body(x_ref, w_ref, ye_ref, o_ref):
    acc = jnp.dot(x_ref[...], w_ref[...], preferred_element_type=jnp.float32)
    o_ref[...] = acc + ye_ref[...]


def _mm2_add(x, w2a, y_exp):
    n = x.shape[0]
    return pl.pallas_call(
        _mm2_add_body,
        grid=(n // _EBLK2,),
        in_specs=[
            pl.BlockSpec((_EBLK2, _D), lambda i: (i, 0)),
            pl.BlockSpec((_D, _D), lambda i: (0, 0)),
            pl.BlockSpec((_EBLK2, _D), lambda i: (i, 0)),
        ],
        out_specs=pl.BlockSpec((_EBLK2, _D), lambda i: (i, 0)),
        out_shape=jax.ShapeDtypeStruct((n, _D), jnp.float32),
    )(x, w2a, y_exp)


_NBLK = 2000


def _mm_small_body(zm_ref, w_ref, y_ref):
    y_ref[...] = jnp.dot(zm_ref[...], w_ref[...], preferred_element_type=jnp.float32)


def _mm_small(zmax, w2b):
    return pl.pallas_call(
        _mm_small_body,
        grid=(_N_NODES // _NBLK,),
        in_specs=[
            pl.BlockSpec((_NBLK, _D), lambda i: (i, 0)),
            pl.BlockSpec((_D, _D), lambda i: (0, 0)),
        ],
        out_specs=pl.BlockSpec((_NBLK, _D), lambda i: (i, 0)),
        out_shape=jax.ShapeDtypeStruct((_N_NODES, _D), jnp.float32),
    )(zmax, w2b)


_CHUNK = _N_EDGES // _NW
_SB = 400
_NREG = _D // 16

_sc_mesh = plsc.VectorSubcoreMesh(core_axis_name="c", subcore_axis_name="s")


_NEG = float("-inf")


@functools.partial(
    pl.kernel,
    out_type=jax.ShapeDtypeStruct((_N_NODES * _D,), jnp.float32),
    mesh=_sc_mesh,
    scratch_types=[
        pltpu.VMEM((_SB,), jnp.int32),
        pltpu.VMEM((_SB,), jnp.int32),
        pltpu.VMEM((_SB * _D,), jnp.float32),
        pltpu.VMEM((_SB * _D,), jnp.float32),
        pltpu.VMEM((16,), jnp.int32),
        pltpu.VMEM((_D,), jnp.float32),
        pltpu.VMEM((_D,), jnp.float32),
        pltpu.VMEM((16,), jnp.int32),
        pltpu.VMEM((16,), jnp.int32),
        pltpu.VMEM((16,), jnp.int32),
        pltpu.VMEM((16,), jnp.int32),
        pltpu.VMEM((_D,), jnp.float32),
        pltpu.SemaphoreType.DMA,
        pltpu.SemaphoreType.DMA,
        pltpu.SemaphoreType.DMA,
        pltpu.SemaphoreType.DMA,
        pltpu.SemaphoreType.DMA,
        pltpu.SemaphoreType.DMA,
    ],
)
def _segmax_kernel(z_hbm, vid_hbm, zmax_hbm, ids0_v, ids1_v, z0_v, z1_v,
                   prev_v, row0, row1, own_v, cur_v, prevg_v, fcnt_v, acc_v,
                   si0, si1, sz0, sz1, sf0, sf1):
    wid = lax.axis_index("s") * _NC + lax.axis_index("c")
    start = wid * _CHUNK
    chunk_end = start + _CHUNK
    _P1 = _CHUNK // _SB
    _PAIRS = _P1 // 2

    @pl.when(wid > 0)
    def _():
        pltpu.sync_copy(vid_hbm.at[pl.ds(start - 16, 16)], prev_v)

    prev0 = jnp.where(wid > 0, prev_v[...][15], -1)

    def flush_slot(row_v, sem, fcnt, cur, a):
        @pl.when(fcnt >= 2)
        def _():
            pltpu.make_async_copy(row_v, zmax_hbm.at[pl.ds(0, _D)], sem).wait()
        for k in range(_NREG):
            row_v[pl.ds(16 * k, 16)] = a[k]
        pltpu.async_copy(row_v, zmax_hbm.at[pl.ds(cur * _D, _D)], sem)

    def one_edge(z_ref, ge, base, eid, c):
        own, cur, prev, fcnt = c[0], c[1], c[2], c[3]
        a = c[4:]
        row = tuple(z_ref[pl.ds(base + 16 * k, 16)] for k in range(_NREG))
        in_chunk = ge < chunk_end
        b = eid != prev
        flush_now = b & (own == 1)

        @pl.when(flush_now)
        def _():
            @pl.when(lax.rem(fcnt, 2) == 0)
            def _():
                flush_slot(row0, sf0, fcnt, cur, a)

            @pl.when(lax.rem(fcnt, 2) == 1)
            def _():
                flush_slot(row1, sf1, fcnt, cur, a)

        new_own = jnp.where(
            b, jnp.where(in_chunk, 1, 0), own
        )
        new_cur = jnp.where(b, eid, cur)
        new_fcnt = fcnt + jnp.where(flush_now, 1, 0)
        new_a = tuple(
            jnp.maximum(jnp.where(b, _NEG, a[k]), row[k])
            for k in range(_NREG)
        )
        return (new_own, new_cur, eid, new_fcnt) + new_a

    own_v[...] = jnp.full((16,), 0, jnp.int32)
    cur_v[...] = jnp.full((16,), -1, jnp.int32)
    prevg_v[...] = jnp.full((16,), prev0, jnp.int32)
    fcnt_v[...] = jnp.full((16,), 0, jnp.int32)

    def scan_block(ids_ref, z_ref, blk):
        own0 = own_v[...][0]
        cur0 = cur_v[...][0]
        prevg0 = prevg_v[...][0]
        fcnt0 = fcnt_v[...][0]
        a0 = tuple(acc_v[pl.ds(16 * k, 16)] for k in range(_NREG))

        def group_step(gi, c):
            idvec = ids_ref[pl.ds(gi * 16, 16)]
            for l in range(16):
                c = one_edge(
                    z_ref, blk + gi * 16 + l, (gi * 16 + l) * _D, idvec[l], c
                )
            return c

        res = lax.fori_loop(
            0, _SB // 16, group_step, (own0, cur0, prevg0, fcnt0) + a0
        )
        own_v[...] = jnp.full((16,), res[0], jnp.int32)
        cur_v[...] = jnp.full((16,), res[1], jnp.int32)
        prevg_v[...] = jnp.full((16,), res[2], jnp.int32)
        fcnt_v[...] = jnp.full((16,), res[3], jnp.int32)
        for k in range(_NREG):
            acc_v[pl.ds(16 * k, 16)] = res[4 + k]

    def fetch(blk, ids_ref, z_ref, sem_i, sem_z):
        pltpu.async_copy(vid_hbm.at[pl.ds(blk, _SB)], ids_ref, sem_i)
        pltpu.async_copy(z_hbm.at[pl.ds(blk * _D, _SB * _D)], z_ref, sem_z)

    def fetch_wait(blk, ids_ref, z_ref, sem_i, sem_z):
        pltpu.make_async_copy(
            vid_hbm.at[pl.ds(blk, _SB)], ids_ref, sem_i).wait()
        pltpu.make_async_copy(
            z_hbm.at[pl.ds(blk * _D, _SB * _D)], z_ref, sem_z).wait()

    fetch(start, ids0_v, z0_v, si0, sz0)
    fetch(start + _SB, ids1_v, z1_v, si1, sz1)

    def pair_body(i, carry):
        b0 = start + (2 * i) * _SB
        b1 = b0 + _SB
        fetch_wait(b0, ids0_v, z0_v, si0, sz0)
        scan_block(ids0_v, z0_v, b0)

        @pl.when(2 * i + 2 < _P1)
        def _():
            fetch(b0 + 2 * _SB, ids0_v, z0_v, si0, sz0)

        fetch_wait(b1, ids1_v, z1_v, si1, sz1)
        scan_block(ids1_v, z1_v, b1)

        @pl.when(2 * i + 3 < _P1)
        def _():
            fetch(b1 + 2 * _SB, ids1_v, z1_v, si1, sz1)

        return carry

    lax.fori_loop(0, _PAIRS, pair_body, 0)
    if _P1 % 2 == 1:
        bt = start + (_P1 - 1) * _SB
        fetch_wait(bt, ids0_v, z0_v, si0, sz0)
        scan_block(ids0_v, z0_v, bt)

    nspill = (_N_EDGES - chunk_end) // _SB

    def spill_body(g, carry):
        blk = chunk_end + g * _SB

        @pl.when(own_v[...][0] == 1)
        def _():
            pltpu.sync_copy(vid_hbm.at[pl.ds(blk, _SB)], ids0_v)
            pltpu.sync_copy(z_hbm.at[pl.ds(blk * _D, _SB * _D)], z0_v)
            scan_block(ids0_v, z0_v, blk)

        return carry

    lax.fori_loop(0, nspill, spill_body, 0)

    own = own_v[...][0]
    cur = cur_v[...][0]

    @pl.when(own == 1)
    def _():
        pltpu.sync_copy(acc_v, zmax_hbm.at[pl.ds(cur * _D, _D)])

    fcnt = fcnt_v[...][0]

    @pl.when(fcnt >= 1)
    def _():
        last = lax.rem(fcnt - 1, 2)

        @pl.when(last == 0)
        def _():
            pltpu.make_async_copy(row0, zmax_hbm.at[pl.ds(0, _D)], sf0).wait()

        @pl.when(last == 1)
        def _():
            pltpu.make_async_copy(row1, zmax_hbm.at[pl.ds(0, _D)], sf1).wait()

    @pl.when(fcnt >= 2)
    def _():
        last2 = lax.rem(fcnt - 2, 2)

        @pl.when(last2 == 0)
        def _():
            pltpu.make_async_copy(row0, zmax_hbm.at[pl.ds(0, _D)], sf0).wait()

        @pl.when(last2 == 1)
        def _():
            pltpu.make_async_copy(row1, zmax_hbm.at[pl.ds(0, _D)], sf1).wait()


_GB = 400


def _make_pump(n_slice):
    chunk = n_slice // _NW
    tpw = chunk // _GB
    pairs = tpw // 2

    @functools.partial(
        pl.kernel,
        out_type=jax.ShapeDtypeStruct((n_slice, _D), jnp.float32),
        mesh=_sc_mesh,
        scratch_types=[
            pltpu.VMEM((_GB,), jnp.int32),
            pltpu.VMEM((_GB,), jnp.int32),
            pltpu.VMEM((_GB, _D), jnp.float32),
            pltpu.VMEM((_GB, _D), jnp.float32),
            pltpu.SemaphoreType.DMA,
            pltpu.SemaphoreType.DMA,
            pltpu.SemaphoreType.DMA,
            pltpu.SemaphoreType.DMA,
        ],
    )
    def _pump(vid_hbm, y_hbm, out_hbm, idx0, idx1, rows0, rows1,
              sg0, sg1, sw0, sw1):
        wid = lax.axis_index("s") * _NC + lax.axis_index("c")
        start = wid * chunk

        def pair(i, carry):
            b0 = start + (2 * i) * _GB
            b1 = start + (2 * i + 1) * _GB

            @pl.when(i > 0)
            def _():
                pltpu.make_async_copy(
                    rows0, out_hbm.at[pl.ds(b0, _GB)], sw0).wait()
                pltpu.make_async_copy(
                    rows1, out_hbm.at[pl.ds(b1, _GB)], sw1).wait()

            pltpu.sync_copy(vid_hbm.at[pl.ds(b0, _GB)], idx0)
            pltpu.async_copy(y_hbm.at[idx0], rows0, sg0)
            pltpu.sync_copy(vid_hbm.at[pl.ds(b1, _GB)], idx1)
            pltpu.async_copy(y_hbm.at[idx1], rows1, sg1)

            pltpu.make_async_copy(y_hbm.at[idx0], rows0, sg0).wait()
            pltpu.async_copy(rows0, out_hbm.at[pl.ds(b0, _GB)], sw0)
            pltpu.make_async_copy(y_hbm.at[idx1], rows1, sg1).wait()
            pltpu.async_copy(rows1, out_hbm.at[pl.ds(b1, _GB)], sw1)
            return carry

        lax.fori_loop(0, pairs, pair, 0)
        if tpw % 2 == 1:
            bt = start + (tpw - 1) * _GB
            if pairs > 0:
                pltpu.make_async_copy(
                    rows0, out_hbm.at[pl.ds(bt - 2 * _GB, _GB)], sw0).wait()
            pltpu.sync_copy(vid_hbm.at[pl.ds(bt, _GB)], idx0)
            pltpu.async_copy(y_hbm.at[idx0], rows0, sg0)
            pltpu.make_async_copy(y_hbm.at[idx0], rows0, sg0).wait()
            pltpu.async_copy(rows0, out_hbm.at[pl.ds(bt, _GB)], sw0)
            if pairs > 0:
                pltpu.make_async_copy(
                    rows1, out_hbm.at[pl.ds(bt - _GB, _GB)], sw1).wait()
            pltpu.make_async_copy(rows0, out_hbm.at[pl.ds(bt, _GB)], sw0).wait()
        else:
            end0 = start + (tpw - 2) * _GB
            end1 = start + (tpw - 1) * _GB
            pltpu.make_async_copy(rows0, out_hbm.at[pl.ds(end0, _GB)], sw0).wait()
            pltpu.make_async_copy(rows1, out_hbm.at[pl.ds(end1, _GB)], sw1).wait()

    return _pump


_N_SLICES = 1
_SLICE = _N_EDGES // _N_SLICES
_pump_slice = _make_pump(_SLICE)


def kernel(x, vertex_id, W1, b1, W2):
    w2a = W2[:_D]
    w2b = W2[_D:]
    z = _mm1(x, W1, b1.reshape(1, _D))
    zmax_flat = _segmax_kernel(z.reshape(-1), vertex_id)
    y = _mm_small(zmax_flat.reshape(_N_NODES, _D), w2b)
    outs = []
    for s in range(_N_SLICES):
        lo = s * _SLICE
        y_exp = _pump_slice(vertex_id[lo:lo + _SLICE], y)
        outs.append(_mm2_add(x[lo:lo + _SLICE], w2a, y_exp))
    if _N_SLICES == 1:
        return outs[0]
    return jnp.concatenate(outs, axis=0)
